# baseline (speedup 1.0000x reference)
import jax
import jax.numpy as jnp
from jax import lax
from jax.experimental import pallas as pl
from jax.experimental.pallas import tpu as pltpu

DEPTH_CONST = 1000
ROWS_PER_BLOCK = 256


def _body(onoff_ref, x_ref, out_ref):
    idx = x_ref[0, 0, :]
    cols = lax.broadcasted_iota(jnp.int32, (ROWS_PER_BLOCK, DEPTH_CONST), 1)
    oh = cols == idx[:, None]
    out_ref[0] = jnp.where(oh, onoff_ref[0, 0], onoff_ref[0, 1])


def kernel(x, on_value, off_value):
    B, S = x.shape
    n = B * S
    g = n // ROWS_PER_BLOCK
    xf = x.reshape(g, 1, ROWS_PER_BLOCK)
    onoff = jnp.stack([on_value, off_value]).reshape(1, 2)
    out = pl.pallas_call(
        _body,
        grid=(g,),
        in_specs=[
            pl.BlockSpec(memory_space=pltpu.SMEM),
            pl.BlockSpec((1, 1, ROWS_PER_BLOCK), lambda i: (i, 0, 0)),
        ],
        out_specs=pl.BlockSpec((1, ROWS_PER_BLOCK, DEPTH_CONST), lambda i: (i, 0, 0)),
        out_shape=jax.ShapeDtypeStruct((g, ROWS_PER_BLOCK, DEPTH_CONST), jnp.float32),
    )(onoff, xf)
    return out.reshape(B, S, DEPTH_CONST)


# speedup vs baseline: 1.5686x; 1.5686x over previous
import jax
import jax.numpy as jnp
from jax import lax
from jax.experimental import pallas as pl
from jax.experimental.pallas import tpu as pltpu

DEPTH_CONST = 1000
BATCH_BLOCK = 16


def _body(onoff_ref, x_ref, out_ref):
    bb, s = x_ref.shape
    cols = lax.broadcasted_iota(jnp.int32, (bb, s, DEPTH_CONST), 2)
    oh = cols == x_ref[...][:, :, None]
    out_ref[...] = jnp.where(oh, onoff_ref[0, 0], onoff_ref[0, 1])


def kernel(x, on_value, off_value):
    B, S = x.shape
    g = B // BATCH_BLOCK
    onoff = jnp.stack([on_value, off_value]).reshape(1, 2)
    out = pl.pallas_call(
        _body,
        grid=(g,),
        in_specs=[
            pl.BlockSpec(memory_space=pltpu.SMEM),
            pl.BlockSpec((BATCH_BLOCK, S), lambda i: (i, 0)),
        ],
        out_specs=pl.BlockSpec((BATCH_BLOCK, S, DEPTH_CONST), lambda i: (i, 0, 0)),
        out_shape=jax.ShapeDtypeStruct((B, S, DEPTH_CONST), jnp.float32),
    )(onoff, x)
    return out
